# Initial kernel scaffold; baseline (speedup 1.0000x reference)
import jax
import jax.numpy as jnp
from jax.experimental import pallas as pl


def kernel(x_movie, x_director, x_actor, ei_md, ei_dm, ei_ma, ei_am, proj_W_movie, proj_b_movie, proj_W_director, proj_b_director, proj_W_actor, proj_b_actor, att_src_md, att_dst_md, att_src_dm, att_dst_dm, att_src_ma, att_dst_ma, att_src_am, att_dst_am, k_W, k_b, q, lin_W, lin_b):
    raise NotImplementedError("write your pallas kernel here")


# speedup vs baseline: 52.7111x; 52.7111x over previous
import functools

import jax
import jax.numpy as jnp
from jax import lax
from jax.experimental import pallas as pl
from jax.experimental.pallas import tpu as pltpu
from jax.experimental.pallas import tpu_sc as plsc

H = 8
D = 16
HID = 128
N = 10000
E = 320000
NCLS = 16

NC = 2
NS = 16
NW = NC * NS
EW = E // NW
K = 80
NCHUNK = EW // K
NP = 10240
RPT = NP // NS
RB = 64

BN = 1000


def _proj_body(x_ref, w_ref, b_ref, am_ref, h_ref, al_ref, cm_ref):
    j = pl.program_id(1)
    x = x_ref[0]
    h = jnp.dot(x, w_ref[0], preferred_element_type=jnp.float32) + b_ref[0, 0]
    h_ref[0] = h
    al = jnp.dot(h, am_ref[0], preferred_element_type=jnp.float32)
    al_ref[0] = al

    @pl.when(j == 0)
    def _():
        cm_ref[...] = jnp.full_like(cm_ref[...], -jnp.inf)

    cm_ref[...] = jnp.maximum(cm_ref[...], al.max(axis=0)[None, None])


def _kernel_a(xs, Ws, bs, Ams):
    return pl.pallas_call(
        _proj_body,
        grid=(3, N // BN),
        in_specs=[
            pl.BlockSpec((1, BN, HID), lambda t, j: (t, j, 0)),
            pl.BlockSpec((1, HID, HID), lambda t, j: (t, 0, 0)),
            pl.BlockSpec((1, 1, HID), lambda t, j: (t, 0, 0)),
            pl.BlockSpec((1, HID, 2 * D), lambda t, j: (t, 0, 0)),
        ],
        out_specs=[
            pl.BlockSpec((1, BN, HID), lambda t, j: (t, j, 0)),
            pl.BlockSpec((1, BN, 2 * D), lambda t, j: (t, j, 0)),
            pl.BlockSpec((1, 1, 2 * D), lambda t, j: (t, 0, 0)),
        ],
        out_shape=[
            jax.ShapeDtypeStruct((3, N, HID), jnp.float32),
            jax.ShapeDtypeStruct((3, N, 2 * D), jnp.float32),
            jax.ShapeDtypeStruct((3, 1, 2 * D), jnp.float32),
        ],
    )(xs, Ws, bs, Ams)


_sc_mesh = plsc.VectorSubcoreMesh(
    core_axis_name="c", subcore_axis_name="s", num_cores=NC, num_subcores=NS)


@functools.partial(
    pl.kernel,
    out_type=[
        jax.ShapeDtypeStruct((E, D), jnp.float32),
        jax.ShapeDtypeStruct((NC, NP, D), jnp.float32),
    ],
    mesh=_sc_mesh,
    compiler_params=pltpu.CompilerParams(use_tc_tiling_on_sc=False),
    scratch_types=[
        pltpu.VMEM_SHARED((NP, D), jnp.float32),
        pltpu.VMEM((K,), jnp.int32),
        pltpu.VMEM((K,), jnp.int32),
        pltpu.VMEM((K, D), jnp.float32),
        pltpu.VMEM((K, D), jnp.float32),
        pltpu.VMEM((K, D), jnp.float32),
        pltpu.VMEM((D,), jnp.float32),
        pltpu.VMEM((RPT, D), jnp.float32),
        pltpu.SemaphoreType.DMA,
        pltpu.SemaphoreType.DMA,
    ],
)
def _edge_softmax_den(src_hbm, dst_hbm, als_hbm, ald_hbm, m_hbm, z_hbm,
                      ex_hbm, denp_hbm,
                      den_sh, sidx, didx, asr, adr, exb, m_v, dtmp,
                      sem1, sem2):
    cid = lax.axis_index("c")
    sid = lax.axis_index("s")
    wid = sid * NC + cid
    r0 = sid * RPT
    pltpu.sync_copy(z_hbm.at[pl.ds(r0, RPT)], den_sh.at[pl.ds(r0, RPT)])
    pltpu.sync_copy(m_hbm, m_v)
    plsc.subcore_barrier()
    base0 = wid * EW

    def chunk(i, carry):
        base = base0 + i * K
        pltpu.sync_copy(src_hbm.at[pl.ds(base, K)], sidx)
        pltpu.sync_copy(dst_hbm.at[pl.ds(base, K)], didx)
        cp1 = pltpu.async_copy(als_hbm.at[sidx], asr, sem1)
        cp2 = pltpu.async_copy(ald_hbm.at[didx], adr, sem2)
        cp1.wait()
        cp2.wait()
        mv = m_v[...]
        for k in range(K):
            a = asr[k, :] + adr[k, :]
            a = jnp.maximum(a, 0.2 * a)
            exb[k, :] = jnp.exp(a - mv)
        pltpu.sync_copy(exb, ex_hbm.at[pl.ds(base, K)])
        pltpu.sync_copy(exb, den_sh.at[didx], add=True)
        return carry

    lax.fori_loop(0, NCHUNK, chunk, 0)
    plsc.subcore_barrier()
    pltpu.sync_copy(den_sh.at[pl.ds(r0, RPT)], dtmp)
    pltpu.sync_copy(dtmp, denp_hbm.at[cid, pl.ds(r0, RPT)])


@functools.partial(
    pl.kernel,
    out_type=jax.ShapeDtypeStruct((NC, NP, HID), jnp.float32),
    mesh=_sc_mesh,
    compiler_params=pltpu.CompilerParams(use_tc_tiling_on_sc=False),
    scratch_types=[
        pltpu.VMEM_SHARED((NP, HID), jnp.float32),
        pltpu.VMEM((K,), jnp.int32),
        pltpu.VMEM((K,), jnp.int32),
        pltpu.VMEM((K, D), jnp.float32),
        pltpu.VMEM((K, HID), jnp.float32),
        pltpu.VMEM((RB, HID), jnp.float32),
        pltpu.VMEM((RB, D), jnp.float32),
        pltpu.SemaphoreType.DMA,
    ],
)
def _edge_message(src_hbm, dst_hbm, ex_hbm, hsrc_hbm, inv_hbm, z_hbm,
                  accp_hbm,
                  acc_sh, sidx, didx, exb, rows, rtmp, invb, sem):
    cid = lax.axis_index("c")
    sid = lax.axis_index("s")
    wid = sid * NC + cid
    r0 = sid * RPT
    pltpu.sync_copy(z_hbm.at[pl.ds(r0, RPT)], acc_sh.at[pl.ds(r0, RPT)])
    plsc.subcore_barrier()
    base0 = wid * EW

    def chunk(i, carry):
        base = base0 + i * K
        pltpu.sync_copy(src_hbm.at[pl.ds(base, K)], sidx)
        pltpu.sync_copy(dst_hbm.at[pl.ds(base, K)], didx)
        pltpu.sync_copy(ex_hbm.at[pl.ds(base, K)], exb)
        pltpu.async_copy(hsrc_hbm.at[sidx], rows, sem).wait()
        for k in range(K):
            exv = exb[k, :]
            for h in range(H):
                w = exv[h]
                rows[k, pl.ds(h * D, D)] = rows[k, pl.ds(h * D, D)] * w
        pltpu.sync_copy(rows, acc_sh.at[didx], add=True)
        return carry

    lax.fori_loop(0, NCHUNK, chunk, 0)
    plsc.subcore_barrier()

    def rblock(j, carry):
        rr = r0 + j * RB
        pltpu.sync_copy(acc_sh.at[pl.ds(rr, RB)], rtmp)
        pltpu.sync_copy(inv_hbm.at[pl.ds(rr, RB)], invb)
        for r in range(RB):
            iv = invb[r, :]
            for h in range(H):
                w = iv[h]
                rtmp[r, pl.ds(h * D, D)] = rtmp[r, pl.ds(h * D, D)] * w
        pltpu.sync_copy(rtmp, accp_hbm.at[cid, pl.ds(rr, RB)])
        return carry

    lax.fori_loop(0, RPT // RB, rblock, 0)


def _invden_body(d_ref, o_ref):
    s = d_ref[0, 0] + d_ref[0, 1]
    o_ref[0] = 1.0 / (s + 1e-16)


def _kernel_d(dall):
    rows = NP * D // 128
    return pl.pallas_call(
        _invden_body,
        grid=(2,),
        in_specs=[pl.BlockSpec((1, NC, rows, 128), lambda r: (r, 0, 0, 0))],
        out_specs=pl.BlockSpec((1, rows, 128), lambda r: (r, 0, 0)),
        out_shape=jax.ShapeDtypeStruct((2, rows, 128), jnp.float32),
    )(dall)


def _semscore_body(ap_ref, kw_ref, kb_ref, stk_ref, ss_ref):
    j = pl.program_id(1)
    p = jnp.maximum(ap_ref[0, 0] + ap_ref[0, 1], 0.0)
    stk_ref[0] = p
    t = jnp.tanh(
        jnp.dot(p, kw_ref[...], preferred_element_type=jnp.float32)
        + kb_ref[...])

    @pl.when(j == 0)
    def _():
        ss_ref[...] = jnp.zeros_like(ss_ref[...])

    ss_ref[...] = ss_ref[...] + t.sum(axis=0)[None, None]


def _kernel_c1(ap, kw, kb):
    return pl.pallas_call(
        _semscore_body,
        grid=(2, N // BN),
        in_specs=[
            pl.BlockSpec((1, NC, BN, HID), lambda r, j: (r, 0, j, 0)),
            pl.BlockSpec((HID, HID), lambda r, j: (0, 0)),
            pl.BlockSpec((HID,), lambda r, j: (0,)),
        ],
        out_specs=[
            pl.BlockSpec((1, BN, HID), lambda r, j: (r, j, 0)),
            pl.BlockSpec((1, 1, HID), lambda r, j: (r, 0, 0)),
        ],
        out_shape=[
            jax.ShapeDtypeStruct((2, N, HID), jnp.float32),
            jax.ShapeDtypeStruct((2, 1, HID), jnp.float32),
        ],
    )(ap, kw, kb)


def _head_body(stk_ref, ss_ref, q_ref, lw_ref, lb_ref, o_ref):
    ss = ss_ref[...]
    qv = q_ref[...]
    s0 = (qv * ss[0]).sum() / N
    s1 = (qv * ss[1]).sum() / N
    m = jnp.maximum(s0, s1)
    e0 = jnp.exp(s0 - m)
    e1 = jnp.exp(s1 - m)
    den = e0 + e1
    comb = stk_ref[0] * (e0 / den) + stk_ref[1] * (e1 / den)
    o_ref[...] = (
        jnp.dot(comb, lw_ref[...], preferred_element_type=jnp.float32)
        + lb_ref[...])


def _kernel_c2(stk, ss, q, lw, lb):
    return pl.pallas_call(
        _head_body,
        grid=(N // BN,),
        in_specs=[
            pl.BlockSpec((2, BN, HID), lambda j: (0, j, 0)),
            pl.BlockSpec((2, HID), lambda j: (0, 0)),
            pl.BlockSpec((HID,), lambda j: (0,)),
            pl.BlockSpec((HID, NCLS), lambda j: (0, 0)),
            pl.BlockSpec((NCLS,), lambda j: (0,)),
        ],
        out_specs=pl.BlockSpec((BN, NCLS), lambda j: (j, 0)),
        out_shape=jax.ShapeDtypeStruct((N, NCLS), jnp.float32),
    )(stk, ss, q, lw, lb)


def _expand(att):
    eye = jnp.eye(H, dtype=att.dtype)
    return (att[:, :, None] * eye[:, None, :]).reshape(HID, H)


def _leaky(v):
    return jnp.maximum(v, 0.2 * v)


def kernel(x_movie, x_director, x_actor, ei_md, ei_dm, ei_ma, ei_am,
           proj_W_movie, proj_b_movie, proj_W_director, proj_b_director,
           proj_W_actor, proj_b_actor, att_src_md, att_dst_md, att_src_dm,
           att_dst_dm, att_src_ma, att_dst_ma, att_src_am, att_dst_am,
           k_W, k_b, q, lin_W, lin_b):
    xs = jnp.stack([x_movie, x_director, x_actor])
    Ws = jnp.stack([proj_W_movie, proj_W_director, proj_W_actor])
    bs = jnp.stack(
        [proj_b_movie, proj_b_director, proj_b_actor]).reshape(3, 1, HID)

    z8 = jnp.zeros((HID, H), jnp.float32)
    e_dst_dm = _expand(att_dst_dm)
    e_dst_am = _expand(att_dst_am)
    e_src_dm = _expand(att_src_dm)
    e_src_am = _expand(att_src_am)
    Ams = jnp.stack([
        jnp.concatenate([e_dst_dm, e_dst_dm, e_dst_am, e_dst_am], axis=1),
        jnp.concatenate([e_src_dm, e_src_dm, z8, z8], axis=1),
        jnp.concatenate([z8, z8, e_src_am, e_src_am], axis=1),
    ])

    h, alph, cm = _kernel_a(xs, Ws, bs, Ams)
    cm = cm[:, 0]

    als_dm = alph[1, :, 0:16]
    ald_dm = alph[0, :, 0:16]
    als_am = alph[2, :, 16:32]
    ald_am = alph[0, :, 16:32]
    M_dm = _leaky(cm[1, 0:16] + cm[0, 0:16])
    M_am = _leaky(cm[2, 16:32] + cm[0, 16:32])

    z16 = jnp.zeros((NP, D), jnp.float32)
    z128 = jnp.zeros((NP, HID), jnp.float32)

    ex_dm, denp_dm = _edge_softmax_den(
        ei_dm[0], ei_dm[1], als_dm, ald_dm, M_dm, z16)
    ex_am, denp_am = _edge_softmax_den(
        ei_am[0], ei_am[1], als_am, ald_am, M_am, z16)

    dall = jnp.stack([denp_dm, denp_am]).reshape(2, NC, NP * D // 128, 128)
    inv = _kernel_d(dall).reshape(2, NP, D)

    accp_dm = _edge_message(
        ei_dm[0], ei_dm[1], ex_dm, h[1], inv[0], z128)
    accp_am = _edge_message(
        ei_am[0], ei_am[1], ex_am, h[2], inv[1], z128)

    stk, ss = _kernel_c1(jnp.stack([accp_dm, accp_am]), k_W, k_b)
    return _kernel_c2(stk, ss.reshape(2, HID), q, lin_W, lin_b)


# speedup vs baseline: 99.9681x; 1.8965x over previous
import functools

import jax
import jax.numpy as jnp
from jax import lax
from jax.experimental import pallas as pl
from jax.experimental.pallas import tpu as pltpu
from jax.experimental.pallas import tpu_sc as plsc

H = 8
D = 16
HID = 128
N = 10000
E = 320000
NCLS = 16

NC = 2
NS = 16
NW = NC * NS
EW = E // NW
K = 80
NCHUNK = EW // K
NSTEP = (NCHUNK + 3) // 4
NP = 10240
RPT = NP // NS
RB = 64

BN = 1000


def _proj_body(x_ref, w_ref, b_ref, am_ref, h_ref, al_ref, cm_ref):
    j = pl.program_id(1)
    x = x_ref[0]
    h = jnp.dot(x, w_ref[0], preferred_element_type=jnp.float32) + b_ref[0, 0]
    h_ref[0] = h
    al = jnp.dot(h, am_ref[0], preferred_element_type=jnp.float32)
    al_ref[0] = al

    @pl.when(j == 0)
    def _():
        cm_ref[...] = jnp.full_like(cm_ref[...], -jnp.inf)

    cm_ref[...] = jnp.maximum(cm_ref[...], al.max(axis=0)[None, None])


def _kernel_a(xs, Ws, bs, Ams):
    return pl.pallas_call(
        _proj_body,
        grid=(3, N // BN),
        in_specs=[
            pl.BlockSpec((1, BN, HID), lambda t, j: (t, j, 0)),
            pl.BlockSpec((1, HID, HID), lambda t, j: (t, 0, 0)),
            pl.BlockSpec((1, 1, HID), lambda t, j: (t, 0, 0)),
            pl.BlockSpec((1, HID, 2 * D), lambda t, j: (t, 0, 0)),
        ],
        out_specs=[
            pl.BlockSpec((1, BN, HID), lambda t, j: (t, j, 0)),
            pl.BlockSpec((1, BN, 2 * D), lambda t, j: (t, j, 0)),
            pl.BlockSpec((1, 1, 2 * D), lambda t, j: (t, 0, 0)),
        ],
        out_shape=[
            jax.ShapeDtypeStruct((3, N, HID), jnp.float32),
            jax.ShapeDtypeStruct((3, N, 2 * D), jnp.float32),
            jax.ShapeDtypeStruct((3, 1, 2 * D), jnp.float32),
        ],
    )(xs, Ws, bs, Ams)


_sc_mesh = plsc.VectorSubcoreMesh(
    core_axis_name="c", subcore_axis_name="s", num_cores=NC, num_subcores=NS)


@functools.partial(
    pl.kernel,
    out_type=[
        jax.ShapeDtypeStruct((E, D), jnp.float32),
        jax.ShapeDtypeStruct((NC, NP, D), jnp.float32),
    ],
    mesh=_sc_mesh,
    compiler_params=pltpu.CompilerParams(use_tc_tiling_on_sc=False),
    scratch_types=[
        pltpu.VMEM_SHARED((NP, D), jnp.float32),
        pltpu.VMEM((K,), jnp.int32),
        pltpu.VMEM((K,), jnp.int32),
        pltpu.VMEM((K,), jnp.int32),
        pltpu.VMEM((K,), jnp.int32),
        pltpu.VMEM((K,), jnp.int32),
        pltpu.VMEM((K,), jnp.int32),
        pltpu.VMEM((K,), jnp.int32),
        pltpu.VMEM((K,), jnp.int32),
        pltpu.VMEM((K, D), jnp.float32),
        pltpu.VMEM((K, D), jnp.float32),
        pltpu.VMEM((K, D), jnp.float32),
        pltpu.VMEM((K, D), jnp.float32),
        pltpu.VMEM((K, D), jnp.float32),
        pltpu.VMEM((D,), jnp.float32),
        pltpu.VMEM((RPT, D), jnp.float32),
        pltpu.SemaphoreType.DMA,
        pltpu.SemaphoreType.DMA,
        pltpu.SemaphoreType.DMA,
        pltpu.SemaphoreType.DMA,
        pltpu.SemaphoreType.DMA,
        pltpu.SemaphoreType.DMA,
    ],
)
def _edge_softmax_den(src_hbm, dst_hbm, als_hbm, ald_hbm, m_hbm, z_hbm,
                      ex_hbm, denp_hbm,
                      den_sh, si0, si1, si2, si3, di0, di1, di2, di3,
                      asr0, asr1, adr0, adr1, exb, m_v, dtmp,
                      mi0, mi1, mi2, mi3, mg0, mg1):
    cid = lax.axis_index("c")
    sid = lax.axis_index("s")
    wid = sid * NC + cid
    r0 = sid * RPT
    pltpu.sync_copy(z_hbm.at[pl.ds(r0, RPT)], den_sh.at[pl.ds(r0, RPT)])
    pltpu.sync_copy(m_hbm, m_v)
    plsc.subcore_barrier()
    base0 = wid * EW

    sis = [si0, si1, si2, si3]
    dis = [di0, di1, di2, di3]
    asrs = [asr0, asr1]
    adrs = [adr0, adr1]
    mis = [mi0, mi1, mi2, mi3]
    mgs = [mg0, mg1]

    def idx_cps(c, q):
        base = base0 + c * K
        return (
            pltpu.make_async_copy(src_hbm.at[pl.ds(base, K)], sis[q], mis[q]),
            pltpu.make_async_copy(dst_hbm.at[pl.ds(base, K)], dis[q], mis[q]),
        )

    def gat_cps(q, s):
        return (
            pltpu.make_async_copy(als_hbm.at[sis[q]], asrs[s], mgs[s]),
            pltpu.make_async_copy(ald_hbm.at[dis[q]], adrs[s], mgs[s]),
        )

    for cp in idx_cps(0, 0) + idx_cps(1, 1) + idx_cps(2, 2):
        cp.start()
    for cp in idx_cps(0, 0):
        cp.wait()
    for cp in gat_cps(0, 0):
        cp.start()

    def step(j, carry):
        c0 = j * 4
        for t in range(4):
            c = c0 + t
            q1 = (t + 1) % 4
            s = t % 2
            s1 = (t + 1) % 2

            @pl.when(c + 1 < NCHUNK)
            def _():
                for cp in idx_cps(c + 1, q1):
                    cp.wait()
                for cp in gat_cps(q1, s1):
                    cp.start()

            @pl.when(c + 3 < NCHUNK)
            def _():
                for cp in idx_cps(c + 3, (t + 3) % 4):
                    cp.start()

            @pl.when(c < NCHUNK)
            def _():
                for cp in gat_cps(t, s):
                    cp.wait()
                mv = m_v[...]
                for k in range(K):
                    a = asrs[s][k, :] + adrs[s][k, :]
                    a = jnp.maximum(a, 0.2 * a)
                    exb[k, :] = jnp.exp(a - mv)
                base = base0 + c * K
                pltpu.sync_copy(exb, ex_hbm.at[pl.ds(base, K)])
                pltpu.sync_copy(exb, den_sh.at[dis[t]], add=True)
        return carry

    lax.fori_loop(0, NSTEP, step, 0)
    plsc.subcore_barrier()
    pltpu.sync_copy(den_sh.at[pl.ds(r0, RPT)], dtmp)
    pltpu.sync_copy(dtmp, denp_hbm.at[cid, pl.ds(r0, RPT)])


@functools.partial(
    pl.kernel,
    out_type=jax.ShapeDtypeStruct((NC, NP, HID), jnp.float32),
    mesh=_sc_mesh,
    compiler_params=pltpu.CompilerParams(use_tc_tiling_on_sc=False),
    scratch_types=[
        pltpu.VMEM_SHARED((NP, HID), jnp.float32),
        pltpu.VMEM((K,), jnp.int32),
        pltpu.VMEM((K,), jnp.int32),
        pltpu.VMEM((K,), jnp.int32),
        pltpu.VMEM((K,), jnp.int32),
        pltpu.VMEM((K,), jnp.int32),
        pltpu.VMEM((K,), jnp.int32),
        pltpu.VMEM((K,), jnp.int32),
        pltpu.VMEM((K,), jnp.int32),
        pltpu.VMEM((K, D), jnp.float32),
        pltpu.VMEM((K, D), jnp.float32),
        pltpu.VMEM((K, HID), jnp.float32),
        pltpu.VMEM((K, HID), jnp.float32),
        pltpu.VMEM((RB, HID), jnp.float32),
        pltpu.VMEM((RB, D), jnp.float32),
        pltpu.VMEM((RB, D), jnp.float32),
        pltpu.SemaphoreType.DMA,
        pltpu.SemaphoreType.DMA,
        pltpu.SemaphoreType.DMA,
        pltpu.SemaphoreType.DMA,
        pltpu.SemaphoreType.DMA,
        pltpu.SemaphoreType.DMA,
    ],
)
def _edge_message(src_hbm, dst_hbm, ex_hbm, hsrc_hbm, denp_hbm, z_hbm,
                  accp_hbm,
                  acc_sh, si0, si1, si2, si3, di0, di1, di2, di3,
                  exb0, exb1, rows0, rows1, rtmp, d0b, d1b,
                  mi0, mi1, mi2, mi3, mg0, mg1):
    cid = lax.axis_index("c")
    sid = lax.axis_index("s")
    wid = sid * NC + cid
    r0 = sid * RPT
    pltpu.sync_copy(z_hbm.at[pl.ds(r0, RPT)], acc_sh.at[pl.ds(r0, RPT)])
    plsc.subcore_barrier()
    base0 = wid * EW

    sis = [si0, si1, si2, si3]
    dis = [di0, di1, di2, di3]
    exbs = [exb0, exb1]
    rowss = [rows0, rows1]
    mis = [mi0, mi1, mi2, mi3]
    mgs = [mg0, mg1]

    def idx_cps(c, q):
        base = base0 + c * K
        return (
            pltpu.make_async_copy(src_hbm.at[pl.ds(base, K)], sis[q], mis[q]),
            pltpu.make_async_copy(dst_hbm.at[pl.ds(base, K)], dis[q], mis[q]),
        )

    def gat_cps(c, q, s):
        base = base0 + c * K
        return (
            pltpu.make_async_copy(hsrc_hbm.at[sis[q]], rowss[s], mgs[s]),
            pltpu.make_async_copy(ex_hbm.at[pl.ds(base, K)], exbs[s], mgs[s]),
        )

    for cp in idx_cps(0, 0) + idx_cps(1, 1) + idx_cps(2, 2):
        cp.start()
    for cp in idx_cps(0, 0):
        cp.wait()
    for cp in gat_cps(0, 0, 0):
        cp.start()

    def step(j, carry):
        c0 = j * 4
        for t in range(4):
            c = c0 + t
            q1 = (t + 1) % 4
            s = t % 2
            s1 = (t + 1) % 2

            @pl.when(c + 1 < NCHUNK)
            def _():
                for cp in idx_cps(c + 1, q1):
                    cp.wait()
                for cp in gat_cps(c + 1, q1, s1):
                    cp.start()

            @pl.when(c + 3 < NCHUNK)
            def _():
                for cp in idx_cps(c + 3, (t + 3) % 4):
                    cp.start()

            @pl.when(c < NCHUNK)
            def _():
                for cp in gat_cps(c, t, s):
                    cp.wait()
                for k in range(K):
                    exv = exbs[s][k, :]
                    for h in range(H):
                        w = exv[h]
                        rowss[s][k, pl.ds(h * D, D)] = (
                            rowss[s][k, pl.ds(h * D, D)] * w)
                pltpu.sync_copy(rowss[s], acc_sh.at[dis[t]], add=True)
        return carry

    lax.fori_loop(0, NSTEP, step, 0)
    plsc.subcore_barrier()

    def rblock(jb, carry):
        rr = r0 + jb * RB
        pltpu.sync_copy(acc_sh.at[pl.ds(rr, RB)], rtmp)
        pltpu.sync_copy(denp_hbm.at[0, pl.ds(rr, RB)], d0b)
        pltpu.sync_copy(denp_hbm.at[1, pl.ds(rr, RB)], d1b)
        for r in range(RB):
            iv = 1.0 / (d0b[r, :] + d1b[r, :] + 1e-16)
            for h in range(H):
                w = iv[h]
                rtmp[r, pl.ds(h * D, D)] = rtmp[r, pl.ds(h * D, D)] * w
        pltpu.sync_copy(rtmp, accp_hbm.at[cid, pl.ds(rr, RB)])
        return carry

    lax.fori_loop(0, RPT // RB, rblock, 0)


def _semscore_body(ap_ref, kw_ref, kb_ref, stk_ref, ss_ref):
    j = pl.program_id(1)
    p = jnp.maximum(ap_ref[0, 0] + ap_ref[0, 1], 0.0)
    stk_ref[0] = p
    t = jnp.tanh(
        jnp.dot(p, kw_ref[...], preferred_element_type=jnp.float32)
        + kb_ref[...])

    @pl.when(j == 0)
    def _():
        ss_ref[...] = jnp.zeros_like(ss_ref[...])

    ss_ref[...] = ss_ref[...] + t.sum(axis=0)[None, None]


def _kernel_c1(ap, kw, kb):
    return pl.pallas_call(
        _semscore_body,
        grid=(2, N // BN),
        in_specs=[
            pl.BlockSpec((1, NC, BN, HID), lambda r, j: (r, 0, j, 0)),
            pl.BlockSpec((HID, HID), lambda r, j: (0, 0)),
            pl.BlockSpec((HID,), lambda r, j: (0,)),
        ],
        out_specs=[
            pl.BlockSpec((1, BN, HID), lambda r, j: (r, j, 0)),
            pl.BlockSpec((1, 1, HID), lambda r, j: (r, 0, 0)),
        ],
        out_shape=[
            jax.ShapeDtypeStruct((2, N, HID), jnp.float32),
            jax.ShapeDtypeStruct((2, 1, HID), jnp.float32),
        ],
    )(ap, kw, kb)


def _head_body(stk_ref, ss_ref, q_ref, lw_ref, lb_ref, o_ref):
    ss = ss_ref[...]
    qv = q_ref[...]
    s0 = (qv * ss[0]).sum() / N
    s1 = (qv * ss[1]).sum() / N
    m = jnp.maximum(s0, s1)
    e0 = jnp.exp(s0 - m)
    e1 = jnp.exp(s1 - m)
    den = e0 + e1
    comb = stk_ref[0] * (e0 / den) + stk_ref[1] * (e1 / den)
    o_ref[...] = (
        jnp.dot(comb, lw_ref[...], preferred_element_type=jnp.float32)
        + lb_ref[...])


def _kernel_c2(stk, ss, q, lw, lb):
    return pl.pallas_call(
        _head_body,
        grid=(N // BN,),
        in_specs=[
            pl.BlockSpec((2, BN, HID), lambda j: (0, j, 0)),
            pl.BlockSpec((2, HID), lambda j: (0, 0)),
            pl.BlockSpec((HID,), lambda j: (0,)),
            pl.BlockSpec((HID, NCLS), lambda j: (0, 0)),
            pl.BlockSpec((NCLS,), lambda j: (0,)),
        ],
        out_specs=pl.BlockSpec((BN, NCLS), lambda j: (j, 0)),
        out_shape=jax.ShapeDtypeStruct((N, NCLS), jnp.float32),
    )(stk, ss, q, lw, lb)


def _expand(att):
    eye = jnp.eye(H, dtype=att.dtype)
    return (att[:, :, None] * eye[:, None, :]).reshape(HID, H)


def _leaky(v):
    return jnp.maximum(v, 0.2 * v)


def kernel(x_movie, x_director, x_actor, ei_md, ei_dm, ei_ma, ei_am,
           proj_W_movie, proj_b_movie, proj_W_director, proj_b_director,
           proj_W_actor, proj_b_actor, att_src_md, att_dst_md, att_src_dm,
           att_dst_dm, att_src_ma, att_dst_ma, att_src_am, att_dst_am,
           k_W, k_b, q, lin_W, lin_b):
    xs = jnp.stack([x_movie, x_director, x_actor])
    Ws = jnp.stack([proj_W_movie, proj_W_director, proj_W_actor])
    bs = jnp.stack(
        [proj_b_movie, proj_b_director, proj_b_actor]).reshape(3, 1, HID)

    z8 = jnp.zeros((HID, H), jnp.float32)
    e_dst_dm = _expand(att_dst_dm)
    e_dst_am = _expand(att_dst_am)
    e_src_dm = _expand(att_src_dm)
    e_src_am = _expand(att_src_am)
    Ams = jnp.stack([
        jnp.concatenate([e_dst_dm, e_dst_dm, e_dst_am, e_dst_am], axis=1),
        jnp.concatenate([e_src_dm, e_src_dm, z8, z8], axis=1),
        jnp.concatenate([z8, z8, e_src_am, e_src_am], axis=1),
    ])

    h, alph, cm = _kernel_a(xs, Ws, bs, Ams)
    cm = cm[:, 0]

    als_dm = alph[1, :, 0:16]
    ald_dm = alph[0, :, 0:16]
    als_am = alph[2, :, 16:32]
    ald_am = alph[0, :, 16:32]
    M_dm = _leaky(cm[1, 0:16] + cm[0, 0:16])
    M_am = _leaky(cm[2, 16:32] + cm[0, 16:32])

    z16 = jnp.zeros((NP, D), jnp.float32)
    z128 = jnp.zeros((NP, HID), jnp.float32)

    ex_dm, denp_dm = _edge_softmax_den(
        ei_dm[0], ei_dm[1], als_dm, ald_dm, M_dm, z16)
    ex_am, denp_am = _edge_softmax_den(
        ei_am[0], ei_am[1], als_am, ald_am, M_am, z16)

    accp_dm = _edge_message(
        ei_dm[0], ei_dm[1], ex_dm, h[1], denp_dm, z128)
    accp_am = _edge_message(
        ei_am[0], ei_am[1], ex_am, h[2], denp_am, z128)

    stk, ss = _kernel_c1(jnp.stack([accp_dm, accp_am]), k_W, k_b)
    return _kernel_c2(stk, ss.reshape(2, HID), q, lin_W, lin_b)
